# baseline (speedup 1.0000x reference)
import jax
import jax.numpy as jnp
from jax.experimental import pallas as pl
from jax.experimental.pallas import tpu as pltpu

_B, _N, _D = 8, 1024, 256
_K = 512
_N_ITERS = 10
_BPP = 4


def _kmeans_body(x_ref, labels_ref, centers_ref):
    ones_col = jnp.ones((_N, 1), jnp.bfloat16)
    kiota_col = jax.lax.broadcasted_iota(jnp.int32, (_K, _N), 0)
    xs = [x_ref[i] for i in range(_BPP)]
    x16s = [x.astype(jnp.bfloat16) for x in xs]
    x16es = [jnp.concatenate([x16, ones_col], axis=1) for x16 in x16s]

    def dists(c, x16):
        c2 = jnp.sum(c * c, axis=1, keepdims=True)
        cx = jax.lax.dot_general((-2.0 * c).astype(jnp.bfloat16), x16,
                                 (((1,), (1,)), ((), ())),
                                 preferred_element_type=jnp.float32)
        return c2 + cx

    def step(c, x16, x16e):
        d = dists(c, x16)
        dmin = jnp.min(d, axis=0, keepdims=True)
        mask = (d <= dmin).astype(jnp.bfloat16)
        sums_cnt = jax.lax.dot_general(mask, x16e,
                                       (((1,), (0,)), ((), ())),
                                       preferred_element_type=jnp.float32)

        def tied(_):
            labels = jnp.argmin(d, axis=0, keepdims=True)
            onehot = (labels == kiota_col).astype(jnp.bfloat16)
            return jax.lax.dot_general(onehot, x16e,
                                       (((1,), (0,)), ((), ())),
                                       preferred_element_type=jnp.float32)

        total = jnp.sum(sums_cnt[:, _D])
        sums_cnt = jax.lax.cond(total == float(_N),
                                lambda _: sums_cnt, tied, None)
        sums = sums_cnt[:, :_D]
        counts = sums_cnt[:, _D:]
        newc = sums / jnp.maximum(counts, 1.0)
        return jnp.where(counts > 0, newc, c)

    def body(_, cs):
        return tuple(step(c, x16, x16e)
                     for c, x16, x16e in zip(cs, x16s, x16es))

    cs = jax.lax.fori_loop(0, _N_ITERS, body,
                           tuple(x[:_K, :] for x in xs))
    for i in range(_BPP):
        labels = jnp.argmin(dists(cs[i], x16s[i]), axis=0, keepdims=True)
        labels_ref[i] = labels.astype(jnp.int32)
        centers_ref[i] = cs[i]


def kernel(x):
    labels, centers = pl.pallas_call(
        _kmeans_body,
        grid=(_B // _BPP,),
        in_specs=[pl.BlockSpec((_BPP, _N, _D), lambda b: (b, 0, 0))],
        out_specs=[
            pl.BlockSpec((_BPP, 1, _N), lambda b: (b, 0, 0)),
            pl.BlockSpec((_BPP, _K, _D), lambda b: (b, 0, 0)),
        ],
        out_shape=[
            jax.ShapeDtypeStruct((_B, 1, _N), jnp.int32),
            jax.ShapeDtypeStruct((_B, _K, _D), jnp.float32),
        ],
        compiler_params=pltpu.CompilerParams(
            dimension_semantics=("arbitrary",),
        ),
    )(x)
    return labels.reshape(_B, _N), centers


# speedup vs baseline: 1.2314x; 1.2314x over previous
import jax
import jax.numpy as jnp
from jax.experimental import pallas as pl
from jax.experimental.pallas import tpu as pltpu

_B, _N, _D = 8, 1024, 256
_K = 512
_N_ITERS = 10
_BPP = 8


def _kmeans_body(x_ref, labels_ref, centers_ref):
    ones_col = jnp.ones((_N, 1), jnp.bfloat16)
    kiota_col = jax.lax.broadcasted_iota(jnp.int32, (_K, _N), 0)
    xs = [x_ref[i] for i in range(_BPP)]
    x16s = [x.astype(jnp.bfloat16) for x in xs]
    x16es = [jnp.concatenate([x16, ones_col], axis=1) for x16 in x16s]

    def dists(c, x16):
        c2 = jnp.sum(c * c, axis=1, keepdims=True)
        cx = jax.lax.dot_general((-2.0 * c).astype(jnp.bfloat16), x16,
                                 (((1,), (1,)), ((), ())),
                                 preferred_element_type=jnp.float32)
        return c2 + cx

    def step(c, x16, x16e):
        labels = jnp.argmin(dists(c, x16), axis=0, keepdims=True)
        onehot = (labels == kiota_col).astype(jnp.bfloat16)
        sums_cnt = jax.lax.dot_general(onehot, x16e,
                                       (((1,), (0,)), ((), ())),
                                       preferred_element_type=jnp.float32)
        sums = sums_cnt[:, :_D]
        counts = sums_cnt[:, _D:]
        newc = sums / jnp.maximum(counts, 1.0)
        return jnp.where(counts > 0, newc, c)

    def body(_, cs):
        return tuple(step(c, x16, x16e)
                     for c, x16, x16e in zip(cs, x16s, x16es))

    cs = jax.lax.fori_loop(0, _N_ITERS, body,
                           tuple(x[:_K, :] for x in xs))
    for i in range(_BPP):
        labels = jnp.argmin(dists(cs[i], x16s[i]), axis=0, keepdims=True)
        labels_ref[i] = labels.astype(jnp.int32)
        centers_ref[i] = cs[i]


def kernel(x):
    labels, centers = pl.pallas_call(
        _kmeans_body,
        grid=(_B // _BPP,),
        in_specs=[pl.BlockSpec((_BPP, _N, _D), lambda b: (b, 0, 0))],
        out_specs=[
            pl.BlockSpec((_BPP, 1, _N), lambda b: (b, 0, 0)),
            pl.BlockSpec((_BPP, _K, _D), lambda b: (b, 0, 0)),
        ],
        out_shape=[
            jax.ShapeDtypeStruct((_B, 1, _N), jnp.int32),
            jax.ShapeDtypeStruct((_B, _K, _D), jnp.float32),
        ],
        compiler_params=pltpu.CompilerParams(
            dimension_semantics=("arbitrary",),
        ),
    )(x)
    return labels.reshape(_B, _N), centers


# speedup vs baseline: 1.2565x; 1.0204x over previous
import jax
import jax.numpy as jnp
from jax.experimental import pallas as pl
from jax.experimental.pallas import tpu as pltpu

_B, _N, _D = 8, 1024, 256
_K = 512
_N_ITERS = 10
_BPP = 4


def _make_body(exact):
    def _kmeans_body(x_ref, labels_ref, centers_ref, tied_ref):
        ones_col = jnp.ones((_N, 1), jnp.bfloat16)
        kiota_col = jax.lax.broadcasted_iota(jnp.int32, (_K, _N), 0)
        xs = [x_ref[i] for i in range(_BPP)]
        x16s = [x.astype(jnp.bfloat16) for x in xs]
        x16es = [jnp.concatenate([x16, ones_col], axis=1) for x16 in x16s]

        def dists(c, x16):
            c2 = jnp.sum(c * c, axis=1, keepdims=True)
            cx = jax.lax.dot_general((-2.0 * c).astype(jnp.bfloat16), x16,
                                     (((1,), (1,)), ((), ())),
                                     preferred_element_type=jnp.float32)
            return c2 + cx

        def step(c, x16, x16e):
            d = dists(c, x16)
            if exact:
                labels = jnp.argmin(d, axis=0, keepdims=True)
                onehot = (labels == kiota_col).astype(jnp.bfloat16)
            else:
                dmin = jnp.min(d, axis=0, keepdims=True)
                onehot = (d <= dmin).astype(jnp.bfloat16)
            sums_cnt = jax.lax.dot_general(onehot, x16e,
                                           (((1,), (0,)), ((), ())),
                                           preferred_element_type=jnp.float32)
            sums = sums_cnt[:, :_D]
            counts = sums_cnt[:, _D:]
            newc = sums / jnp.maximum(counts, 1.0)
            return jnp.where(counts > 0, newc, c), jnp.sum(counts)

        def body(_, carry):
            cs, tied = carry
            outs = [step(c, x16, x16e)
                    for c, x16, x16e in zip(cs, x16s, x16es)]
            for _, total in outs:
                tied = jnp.maximum(tied, jnp.abs(total - float(_N)))
            return tuple(c for c, _ in outs), tied

        cs, tied = jax.lax.fori_loop(
            0, _N_ITERS, body,
            (tuple(x[:_K, :] for x in xs), jnp.float32(0.0)))
        for i in range(_BPP):
            labels = jnp.argmin(dists(cs[i], x16s[i]), axis=0, keepdims=True)
            labels_ref[i] = labels.astype(jnp.int32)
            centers_ref[i] = cs[i]
        tied_ref[0] = jnp.broadcast_to(tied, (1, 1))

    return _kmeans_body


def _run(x, exact):
    return pl.pallas_call(
        _make_body(exact),
        grid=(_B // _BPP,),
        in_specs=[pl.BlockSpec((_BPP, _N, _D), lambda b: (b, 0, 0))],
        out_specs=[
            pl.BlockSpec((_BPP, 1, _N), lambda b: (b, 0, 0)),
            pl.BlockSpec((_BPP, _K, _D), lambda b: (b, 0, 0)),
            pl.BlockSpec((1, 1, 1), lambda b: (b, 0, 0)),
        ],
        out_shape=[
            jax.ShapeDtypeStruct((_B, 1, _N), jnp.int32),
            jax.ShapeDtypeStruct((_B, _K, _D), jnp.float32),
            jax.ShapeDtypeStruct((_B // _BPP, 1, 1), jnp.float32),
        ],
        compiler_params=pltpu.CompilerParams(
            dimension_semantics=("arbitrary",),
        ),
    )(x)


def kernel(x):
    labels, centers, tied = _run(x, exact=False)
    labels, centers = jax.lax.cond(
        jnp.any(tied > 0),
        lambda: _run(x, exact=True)[:2],
        lambda: (labels, centers),
    )
    return labels.reshape(_B, _N), centers


# speedup vs baseline: 1.2829x; 1.0211x over previous
import jax
import jax.numpy as jnp
from jax.experimental import pallas as pl
from jax.experimental.pallas import tpu as pltpu

_B, _N, _D = 8, 1024, 256
_K = 512
_N_ITERS = 10
_BPP = 4


def _make_body(exact):
    def _kmeans_body(x_ref, labels_ref, centers_ref, tied_ref):
        ones_col = jnp.ones((_N, 1), jnp.bfloat16)
        kiota_col = jax.lax.broadcasted_iota(jnp.int32, (_K, _N), 0)
        xs = [x_ref[i] for i in range(_BPP)]
        x16s = [x.astype(jnp.bfloat16) for x in xs]
        x16es = [jnp.concatenate([x16.astype(jnp.float32),
                                  ones_col.astype(jnp.float32)], axis=1)
                 for x16 in x16s]

        def dists(c, x16):
            c2 = jnp.sum(c * c, axis=1, keepdims=True)
            cx = jax.lax.dot_general((-2.0 * c).astype(jnp.bfloat16), x16,
                                     (((1,), (1,)), ((), ())),
                                     preferred_element_type=jnp.float32)
            return c2 + cx

        def step(c, x16, x16e):
            d = dists(c, x16)
            if exact:
                labels = jnp.argmin(d, axis=0, keepdims=True)
                onehot = (labels == kiota_col).astype(jnp.float32)
            else:
                dmin = jnp.min(d, axis=0, keepdims=True)
                onehot = (d <= dmin).astype(jnp.float32)
            sums_cnt = jax.lax.dot_general(onehot, x16e,
                                           (((1,), (0,)), ((), ())),
                                           preferred_element_type=jnp.float32)
            sums = sums_cnt[:, :_D]
            counts = sums_cnt[:, _D:]
            newc = sums / jnp.maximum(counts, 1.0)
            return jnp.where(counts > 0, newc, c), jnp.sum(counts)

        def body(_, carry):
            cs, tied = carry
            outs = [step(c, x16, x16e)
                    for c, x16, x16e in zip(cs, x16s, x16es)]
            for _, total in outs:
                tied = jnp.maximum(tied, jnp.abs(total - float(_N)))
            return tuple(c for c, _ in outs), tied

        cs, tied = jax.lax.fori_loop(
            0, _N_ITERS, body,
            (tuple(x[:_K, :] for x in xs), jnp.float32(0.0)))
        for i in range(_BPP):
            labels = jnp.argmin(dists(cs[i], x16s[i]), axis=0, keepdims=True)
            labels_ref[i] = labels.astype(jnp.int32)
            centers_ref[i] = cs[i]
        tied_ref[0] = jnp.broadcast_to(tied, (1, 1))

    return _kmeans_body


def _run(x, exact):
    return pl.pallas_call(
        _make_body(exact),
        grid=(_B // _BPP,),
        in_specs=[pl.BlockSpec((_BPP, _N, _D), lambda b: (b, 0, 0))],
        out_specs=[
            pl.BlockSpec((_BPP, 1, _N), lambda b: (b, 0, 0)),
            pl.BlockSpec((_BPP, _K, _D), lambda b: (b, 0, 0)),
            pl.BlockSpec((1, 1, 1), lambda b: (b, 0, 0)),
        ],
        out_shape=[
            jax.ShapeDtypeStruct((_B, 1, _N), jnp.int32),
            jax.ShapeDtypeStruct((_B, _K, _D), jnp.float32),
            jax.ShapeDtypeStruct((_B // _BPP, 1, 1), jnp.float32),
        ],
        compiler_params=pltpu.CompilerParams(
            dimension_semantics=("arbitrary",),
        ),
    )(x)


def kernel(x):
    labels, centers, tied = _run(x, exact=False)
    labels, centers = jax.lax.cond(
        jnp.any(tied > 0),
        lambda: _run(x, exact=True)[:2],
        lambda: (labels, centers),
    )
    return labels.reshape(_B, _N), centers


# speedup vs baseline: 1.3336x; 1.0395x over previous
import jax
import jax.numpy as jnp
from jax.experimental import pallas as pl
from jax.experimental.pallas import tpu as pltpu

_B, _N, _D = 8, 1024, 256
_K = 512
_N_ITERS = 10
_BPP = 4


def _make_body(exact):
    def _kmeans_body(x_ref, labels_ref, centers_ref, tied_ref, c_scr):
        ones_col = jnp.ones((_N, 1), jnp.bfloat16)
        kiota_col = jax.lax.broadcasted_iota(jnp.int32, (_K, _N), 0)
        xs = [x_ref[i] for i in range(_BPP)]
        x16s = [x.astype(jnp.bfloat16) for x in xs]
        x16es = [jnp.concatenate([x16.astype(jnp.float32),
                                  ones_col.astype(jnp.float32)], axis=1)
                 for x16 in x16s]

        def dists(c, x16):
            c2 = jnp.sum(c * c, axis=1, keepdims=True)
            cx = jax.lax.dot_general((-2.0 * c).astype(jnp.bfloat16), x16,
                                     (((1,), (1,)), ((), ())),
                                     preferred_element_type=jnp.float32)
            return c2 + cx

        def step(c, x16, x16e):
            d = dists(c, x16)
            if exact:
                labels = jnp.argmin(d, axis=0, keepdims=True)
                onehot = (labels == kiota_col).astype(jnp.float32)
            else:
                dmin = jnp.min(d, axis=0, keepdims=True)
                onehot = (d <= dmin).astype(jnp.float32)
            sums_cnt = jax.lax.dot_general(onehot, x16e,
                                           (((1,), (0,)), ((), ())),
                                           preferred_element_type=jnp.float32)
            sums = sums_cnt[:, :_D]
            counts = sums_cnt[:, _D:]
            newc = sums / jnp.maximum(counts, 1.0)
            return jnp.where(counts > 0, newc, c), jnp.sum(counts)

        for i in range(_BPP):
            c_scr[i] = xs[i][:_K, :]

        def body(_, tied):
            for i in range(_BPP):
                newc, total = step(c_scr[i], x16s[i], x16es[i])
                c_scr[i] = newc
                tied = jnp.maximum(tied, jnp.abs(total - float(_N)))
            return tied

        tied = jax.lax.fori_loop(0, _N_ITERS, body, jnp.float32(0.0))
        for i in range(_BPP):
            labels = jnp.argmin(dists(c_scr[i], x16s[i]), axis=0, keepdims=True)
            labels_ref[i] = labels.astype(jnp.int32)
            centers_ref[i] = c_scr[i]
        tied_ref[0] = jnp.broadcast_to(tied, (1, 1))

    return _kmeans_body


def _run(x, exact):
    return pl.pallas_call(
        _make_body(exact),
        grid=(_B // _BPP,),
        in_specs=[pl.BlockSpec((_BPP, _N, _D), lambda b: (b, 0, 0))],
        out_specs=[
            pl.BlockSpec((_BPP, 1, _N), lambda b: (b, 0, 0)),
            pl.BlockSpec((_BPP, _K, _D), lambda b: (b, 0, 0)),
            pl.BlockSpec((1, 1, 1), lambda b: (b, 0, 0)),
        ],
        out_shape=[
            jax.ShapeDtypeStruct((_B, 1, _N), jnp.int32),
            jax.ShapeDtypeStruct((_B, _K, _D), jnp.float32),
            jax.ShapeDtypeStruct((_B // _BPP, 1, 1), jnp.float32),
        ],
        scratch_shapes=[pltpu.VMEM((_BPP, _K, _D), jnp.float32)],
        compiler_params=pltpu.CompilerParams(
            dimension_semantics=("arbitrary",),
        ),
    )(x)


def kernel(x):
    labels, centers, tied = _run(x, exact=False)
    labels, centers = jax.lax.cond(
        jnp.any(tied > 0),
        lambda: _run(x, exact=True)[:2],
        lambda: (labels, centers),
    )
    return labels.reshape(_B, _N), centers


# speedup vs baseline: 1.3869x; 1.0399x over previous
import jax
import jax.numpy as jnp
from jax.experimental import pallas as pl
from jax.experimental.pallas import tpu as pltpu

_B, _N, _D = 8, 1024, 256
_K = 512
_N_ITERS = 10
_BPP = 4


def _make_body(exact):
    def _kmeans_body(x_ref, labels_ref, centers_ref, tied_ref, c_scr):
        ones_col = jnp.ones((_N, 1), jnp.bfloat16)
        kiota_col = jax.lax.broadcasted_iota(jnp.int32, (_K, _N), 0)
        xs = [x_ref[i] for i in range(_BPP)]
        x16s = [x.astype(jnp.bfloat16) for x in xs]
        x16es = [jnp.concatenate([x16.astype(jnp.float32),
                                  ones_col.astype(jnp.float32)], axis=1)
                 for x16 in x16s]

        def dists(c, x16):
            c2 = jnp.sum(c * c, axis=1, keepdims=True)
            cx = jax.lax.dot_general((-2.0 * c).astype(jnp.bfloat16), x16,
                                     (((1,), (1,)), ((), ())),
                                     preferred_element_type=jnp.float32)
            return c2 + cx

        def step(c, x16, x16e):
            d = dists(c, x16)
            if exact:
                labels = jnp.argmin(d, axis=0, keepdims=True)
                onehot = (labels == kiota_col).astype(jnp.float32)
            else:
                dmin = jnp.min(d, axis=0, keepdims=True)
                onehot = (d <= dmin).astype(jnp.float32)
            sums_cnt = jax.lax.dot_general(onehot, x16e,
                                           (((1,), (0,)), ((), ())),
                                           preferred_element_type=jnp.float32)
            sums = sums_cnt[:, :_D]
            counts = sums_cnt[:, _D:]
            newc = sums / jnp.maximum(counts, 1.0)
            return jnp.where(counts > 0, newc, c), jnp.sum(counts)

        for i in range(_BPP):
            c_scr[i] = xs[i][:_K, :]

        def body(tied):
            for i in range(_BPP):
                newc, total = step(c_scr[i], x16s[i], x16es[i])
                c_scr[i] = newc
                tied = jnp.maximum(tied, jnp.abs(total - float(_N)))
            return tied

        if exact:
            tied = jax.lax.fori_loop(0, _N_ITERS, lambda _, t: body(t),
                                     jnp.float32(0.0))
        else:
            tied = jnp.float32(0.0)
            for _ in range(_N_ITERS):
                tied = body(tied)
        for i in range(_BPP):
            labels = jnp.argmin(dists(c_scr[i], x16s[i]), axis=0, keepdims=True)
            labels_ref[i] = labels.astype(jnp.int32)
            centers_ref[i] = c_scr[i]
        tied_ref[0] = jnp.broadcast_to(tied, (1, 1))

    return _kmeans_body


def _run(x, exact):
    return pl.pallas_call(
        _make_body(exact),
        grid=(_B // _BPP,),
        in_specs=[pl.BlockSpec((_BPP, _N, _D), lambda b: (b, 0, 0))],
        out_specs=[
            pl.BlockSpec((_BPP, 1, _N), lambda b: (b, 0, 0)),
            pl.BlockSpec((_BPP, _K, _D), lambda b: (b, 0, 0)),
            pl.BlockSpec((1, 1, 1), lambda b: (b, 0, 0)),
        ],
        out_shape=[
            jax.ShapeDtypeStruct((_B, 1, _N), jnp.int32),
            jax.ShapeDtypeStruct((_B, _K, _D), jnp.float32),
            jax.ShapeDtypeStruct((_B // _BPP, 1, 1), jnp.float32),
        ],
        scratch_shapes=[pltpu.VMEM((_BPP, _K, _D), jnp.float32)],
        compiler_params=pltpu.CompilerParams(
            dimension_semantics=("arbitrary",),
        ),
    )(x)


def kernel(x):
    labels, centers, tied = _run(x, exact=False)
    labels, centers = jax.lax.cond(
        jnp.any(tied > 0),
        lambda: _run(x, exact=True)[:2],
        lambda: (labels, centers),
    )
    return labels.reshape(_B, _N), centers
